# baseline (speedup 1.0000x reference)
import jax
import jax.numpy as jnp
from jax.experimental import pallas as pl
from jax.experimental.pallas import tpu as pltpu

_S = 12
_P = 12
_D3 = 48
_GB = 24
_T = 144


def _block_kernel(x_ref, xp_ref, xn_ref, x8_ref, w_ref, bqkv_ref,
                  wd_ref, bd_ref, wu_ref, bu_ref, o_ref):
    i = pl.program_id(1)
    nb = pl.num_programs(1)
    bf = jnp.bfloat16
    R = _T * _GB

    def dott(a, b):
        return jax.lax.dot_general(a, b, (((1,), (1,)), ((), ())),
                                   preferred_element_type=jnp.float32)

    w = w_ref[...]
    wq = w[0:_D3, :]
    wk = w[_D3:2 * _D3, :]
    wv = w[2 * _D3:, :]
    bq = bqkv_ref[0:1, 0:_D3]
    bk = bqkv_ref[0:1, _D3:2 * _D3]
    bv = bqkv_ref[0:1, 2 * _D3:]
    wd = wd_ref[...]
    wu = wu_ref[...]

    x2 = x_ref[0].reshape(R, _S * _S).astype(bf)
    q = dott(x2, wq) + bq
    v = dott(x2, wv) + bv

    x8 = x8_ref[0, 0, 0:2, 0, :].astype(bf)
    k2 = dott(x8, wk) + bk
    v2 = dott(x8, wv) + bv
    kd = k2[0:1, :] - k2[1:2, :]
    kd2 = jnp.concatenate([kd, -kd], axis=0).astype(bf)
    s2 = dott(q.astype(bf), kd2)
    p2 = jax.nn.sigmoid(s2).astype(bf)
    att = jax.lax.dot_general(p2, v2.astype(bf), (((1,), (0,)), ((), ())),
                              preferred_element_type=jnp.float32)

    xhp = xp_ref[0, 47, 1:3, _GB - 1, :].astype(bf)
    xhn = xn_ref[0, 0, 0:2, 0, :].astype(bf)
    vhp = dott(xhp, wv) + bv
    vhn = dott(xhn, wv) + bv
    qhp = dott(xhp, wq) + bq
    qhn = dott(xhn, wq) + bq
    first = (i == 0)
    last = (i == nb - 1)
    vhp = jnp.where(first, 0.0, vhp)
    vhn = jnp.where(last, 0.0, vhn)

    def att_of(qrow):
        srow = dott(qrow.astype(bf), kd2)
        prow = jax.nn.sigmoid(srow).astype(bf)
        return jax.lax.dot_general(prow, v2.astype(bf),
                                   (((1,), (0,)), ((), ())),
                                   preferred_element_type=jnp.float32)

    def shift3(a3, top_row, bot_row):
        row_p = jnp.concatenate(
            [top_row.reshape(1, 1, _D3), a3[_T - 1:_T, :_GB - 1, :]], axis=1)
        prev = jnp.concatenate([row_p, a3[0:_T - 1]], axis=0)
        row_n = jnp.concatenate(
            [a3[0:1, 1:, :], bot_row.reshape(1, 1, _D3)], axis=1)
        nxt = jnp.concatenate([a3[1:_T], row_n], axis=0)
        return prev, nxt

    def tap(a, c, wgt):
        return dott(a.reshape(R, _D3).astype(bf), wgt[:, :, c]).reshape(
            _T, _GB, _D3)

    v3 = v.reshape(_T, _GB, _D3)
    vprev, vnext = shift3(v3, vhp[1], vhn[0])
    bd = bd_ref[...]
    mid3 = (att.reshape(_T, _GB, _D3) + bd + tap(v3, 1, wd)
            + tap(vprev, 0, wd) + tap(vnext, 2, wd))

    mid_hp = (att_of(qhp[1:2]) + bd + dott(vhp[0:1].astype(bf), wd[:, :, 0])
              + dott(vhp[1:2].astype(bf), wd[:, :, 1])
              + dott(v3[0:1, 0, :].astype(bf), wd[:, :, 2]))
    mid_hn = (att_of(qhn[0:1]) + bd
              + dott(v3[_T - 1:_T, _GB - 1, :].astype(bf), wd[:, :, 0])
              + dott(vhn[0:1].astype(bf), wd[:, :, 1])
              + dott(vhn[1:2].astype(bf), wd[:, :, 2]))
    mid_hp = jnp.where(first, 0.0, mid_hp)
    mid_hn = jnp.where(last, 0.0, mid_hn)

    mprev, mnext = shift3(mid3, mid_hp[0], mid_hn[0])
    out3 = (bu_ref[...] + tap(mid3, 1, wu) + tap(mprev, 0, wu)
            + tap(mnext, 2, wu))
    o_ref[0] = out3


def kernel(input, W_qkv, b_qkv, W_down, b_down, W_up, b_up):
    B, C, H, W = input.shape
    x7 = input.reshape(B, 48, 8, _S, _P, _S, _P)
    x7 = jnp.transpose(x7, (0, 1, 2, 4, 6, 3, 5))
    xu5 = x7.reshape(B, 48, 3, 384, _S * _S)
    nb = 384 // _GB

    wqkv = W_qkv.astype(jnp.bfloat16)
    wd = W_down.astype(jnp.bfloat16)
    wu = W_up.astype(jnp.bfloat16)
    bqkv = b_qkv.reshape(1, 3 * _D3)
    bd = b_down.reshape(1, _D3)
    bu = b_up.reshape(1, _D3)

    def full(shp, nd):
        return pl.BlockSpec(shp, (lambda b, i: (0,) * nd))

    out = pl.pallas_call(
        _block_kernel,
        grid=(B, nb),
        in_specs=[
            pl.BlockSpec((1, 48, 3, _GB, _S * _S), lambda b, i: (b, 0, 0, i, 0)),
            pl.BlockSpec((1, 48, 3, _GB, _S * _S),
                         lambda b, i: (b, 0, 0, jnp.maximum(i - 1, 0), 0)),
            pl.BlockSpec((1, 48, 3, _GB, _S * _S),
                         lambda b, i: (b, 0, 0, jnp.minimum(i + 1, nb - 1), 0)),
            pl.BlockSpec((1, 1, 3, 8, _S * _S), lambda b, i: (b, 0, 0, 0, 0)),
            full((_S * _S, _S * _S), 2),
            full((1, 3 * _D3), 2),
            full((_D3, _D3, 3), 3),
            full((1, _D3), 2),
            full((_D3, _D3, 3), 3),
            full((1, _D3), 2),
        ],
        out_specs=pl.BlockSpec((1, _T, _GB, _D3), lambda b, i: (b, 0, i, 0)),
        out_shape=jax.ShapeDtypeStruct((B, _T, 384, _D3), jnp.float32),
        compiler_params=pltpu.CompilerParams(
            dimension_semantics=("arbitrary", "arbitrary")),
    )(xu5, xu5, xu5, xu5, wqkv, bqkv, wd, bd, wu, bu)

    o6 = out.reshape(B, 48, 3, 128, 3, _D3)
    return jnp.transpose(o6, (0, 4, 1, 3, 2, 5)).reshape(B, C // 3, H, W)


# speedup vs baseline: 2.9459x; 2.9459x over previous
import jax
import jax.numpy as jnp
from jax.experimental import pallas as pl
from jax.experimental.pallas import tpu as pltpu

_S = 12
_P = 12
_D3 = 48
_BM = 3456
_HALO = 8


def _block_kernel(M, x_ref, fh_ref, bh_ref, x01_ref, wq_ref, wk_ref, wv_ref,
                  bq_ref, bk_ref, bv_ref, wd_ref, bd_ref, wu_ref, bu_ref,
                  o_ref):
    i = pl.program_id(1)
    BME = _BM + 2 * _HALO
    bf = jnp.bfloat16

    def dot(a, b):
        return jax.lax.dot_general(a, b, (((1,), (0,)), ((), ())),
                                   preferred_element_type=jnp.float32)

    xc = x_ref[0].astype(bf)
    fh = fh_ref[0, 0].astype(bf)
    bh = bh_ref[0, 0].astype(bf)
    x8 = x01_ref[0].astype(bf)

    wq = wq_ref[...]
    wv = wv_ref[...]

    q_ext = jnp.concatenate([dot(fh, wq), dot(xc, wq), dot(bh, wq)],
                            axis=0) + bq_ref[...]
    v_ext = jnp.concatenate([dot(fh, wv), dot(xc, wv), dot(bh, wv)],
                            axis=0) + bv_ref[...]

    rows = jax.lax.broadcasted_iota(jnp.int32, (BME, _D3), 0)
    gi = i * _BM - _HALO + rows
    valid = (gi >= 0) & (gi < M)
    v_m = jnp.where(valid, v_ext, 0.0)

    k8 = dot(x8, wk_ref[...]) + bk_ref[...]
    v8 = dot(x8, wv) + bv_ref[...]
    kd = k8[0:1, :] - k8[1:2, :]
    kd2 = jnp.concatenate([kd, -kd], axis=0).astype(bf)
    s2 = jax.lax.dot_general(q_ext.astype(bf), kd2, (((1,), (1,)), ((), ())),
                             preferred_element_type=jnp.float32)
    p2 = jax.nn.sigmoid(s2).astype(bf)
    att = dot(p2, v8[0:2, :].astype(bf))

    vb = v_m.astype(bf)
    yd0 = dot(vb, wd_ref[0])
    yd1 = dot(vb, wd_ref[1])
    yd2 = dot(vb, wd_ref[2])
    mid_c = att + bd_ref[...] + yd1
    midv = mid_c[1:BME - 1] + yd0[0:BME - 2] + yd2[2:BME]
    midv = jnp.where(valid[1:BME - 1], midv, 0.0).astype(bf)

    yu0 = dot(midv, wu_ref[0])
    yu1 = dot(midv, wu_ref[1])
    yu2 = dot(midv, wu_ref[2])
    out = (yu0[_HALO - 2:_HALO - 2 + _BM] + yu1[_HALO - 1:_HALO - 1 + _BM]
           + yu2[_HALO:_HALO + _BM] + bu_ref[...])
    o_ref[0] = out


def kernel(input, W_qkv, b_qkv, W_down, b_down, W_up, b_up):
    B, C, H, W = input.shape
    xu = input.reshape(B, C, _S, _P, _S, _P)
    xu = jnp.transpose(xu, (0, 1, 3, 5, 2, 4)).reshape(B, C * _P * _P, _S * _S)
    x = xu.reshape(B, _S * _S, -1, _P * _P)
    x = jnp.transpose(x, (0, 2, 1, 3)).reshape(B, -1, _S * _S)
    M = x.shape[1]
    nb = M // _BM

    xr = x.reshape(B, nb, _BM, _S * _S)
    z8 = jnp.zeros((B, 1, _HALO, _S * _S), x.dtype)
    fh = jnp.concatenate([z8, xr[:, :-1, _BM - _HALO:, :]], axis=1)
    bh = jnp.concatenate([xr[:, 1:, :_HALO, :], z8], axis=1)
    x01 = x[:, :_HALO, :]

    WT = W_qkv.T.astype(jnp.bfloat16)
    wq, wk, wv = WT[:, :_D3], WT[:, _D3:2 * _D3], WT[:, 2 * _D3:]
    bq = b_qkv[:_D3].reshape(1, _D3)
    bk = b_qkv[_D3:2 * _D3].reshape(1, _D3)
    bv = b_qkv[2 * _D3:].reshape(1, _D3)
    wd = jnp.transpose(W_down, (2, 1, 0)).astype(jnp.bfloat16)
    wu = jnp.transpose(W_up, (2, 1, 0)).astype(jnp.bfloat16)
    bd = b_down.reshape(1, _D3)
    bu = b_up.reshape(1, _D3)

    def full(shp, nd):
        return pl.BlockSpec(shp, (lambda b, i: (0,) * nd))

    out = pl.pallas_call(
        lambda *refs: _block_kernel(M, *refs),
        grid=(B, nb),
        in_specs=[
            pl.BlockSpec((1, _BM, _S * _S), lambda b, i: (b, i, 0)),
            pl.BlockSpec((1, 1, _HALO, _S * _S), lambda b, i: (b, i, 0, 0)),
            pl.BlockSpec((1, 1, _HALO, _S * _S), lambda b, i: (b, i, 0, 0)),
            pl.BlockSpec((1, _HALO, _S * _S), lambda b, i: (b, 0, 0)),
            full((_S * _S, _D3), 2),
            full((_S * _S, _D3), 2),
            full((_S * _S, _D3), 2),
            full((1, _D3), 2),
            full((1, _D3), 2),
            full((1, _D3), 2),
            full((3, _D3, _D3), 3),
            full((1, _D3), 2),
            full((3, _D3, _D3), 3),
            full((1, _D3), 2),
        ],
        out_specs=pl.BlockSpec((1, _BM, _D3), lambda b, i: (b, i, 0)),
        out_shape=jax.ShapeDtypeStruct((B, M, _D3), jnp.float32),
        compiler_params=pltpu.CompilerParams(
            dimension_semantics=("arbitrary", "arbitrary")),
    )(x, fh, bh, x01, wq, wk, wv, bq, bk, bv, wd, bd, wu, bu)

    out = out.reshape(B, -1, _S * _S, _P * _P)
    out = jnp.transpose(out, (0, 2, 1, 3))
    return out.reshape(B, -1, _S * _P, _S * _P)


# speedup vs baseline: 3.0298x; 1.0285x over previous
import jax
import jax.numpy as jnp
from jax.experimental import pallas as pl
from jax.experimental.pallas import tpu as pltpu

_S = 12
_P = 12
_D3 = 48
_BM = 3456
_HALO = 8


def _block_kernel(M, x_ref, fh_ref, bh_ref, x01_ref, wq_ref, wk_ref, wv_ref,
                  bq_ref, bk_ref, bv_ref, wd_ref, bd_ref, wu_ref, bu_ref,
                  o_ref):
    i = pl.program_id(1)
    BME = _BM + 2 * _HALO

    def dot(a, b):
        return jax.lax.dot_general(a, b, (((1,), (0,)), ((), ())),
                                   preferred_element_type=jnp.float32)

    xc = x_ref[0]
    fh = fh_ref[0, 0]
    bh = bh_ref[0, 0]
    x8 = x01_ref[0]

    wq = wq_ref[...]
    wv = wv_ref[...]

    q_ext = jnp.concatenate([dot(fh, wq), dot(xc, wq), dot(bh, wq)],
                            axis=0) + bq_ref[...]
    v_ext = jnp.concatenate([dot(fh, wv), dot(xc, wv), dot(bh, wv)],
                            axis=0) + bv_ref[...]

    rows = jax.lax.broadcasted_iota(jnp.int32, (BME, _D3), 0)
    gi = i * _BM - _HALO + rows
    valid = (gi >= 0) & (gi < M)
    v_m = jnp.where(valid, v_ext, 0.0)

    k8 = dot(x8, wk_ref[...]) + bk_ref[...]
    v8 = dot(x8, wv) + bv_ref[...]
    kd = k8[0:1, :] - k8[1:2, :]
    kd2 = jnp.concatenate([kd, -kd], axis=0)
    s2 = jax.lax.dot_general(q_ext, kd2, (((1,), (1,)), ((), ())),
                             preferred_element_type=jnp.float32)
    p2 = jax.nn.sigmoid(s2)
    att = dot(p2, v8[0:2, :])

    vb = v_m
    yd0 = dot(vb, wd_ref[0])
    yd1 = dot(vb, wd_ref[1])
    yd2 = dot(vb, wd_ref[2])
    mid_c = att + bd_ref[...] + yd1
    midv = mid_c[1:BME - 1] + yd0[0:BME - 2] + yd2[2:BME]
    midv = jnp.where(valid[1:BME - 1], midv, 0.0)

    yu0 = dot(midv, wu_ref[0])
    yu1 = dot(midv, wu_ref[1])
    yu2 = dot(midv, wu_ref[2])
    out = (yu0[_HALO - 2:_HALO - 2 + _BM] + yu1[_HALO - 1:_HALO - 1 + _BM]
           + yu2[_HALO:_HALO + _BM] + bu_ref[...])
    o_ref[0] = out


def kernel(input, W_qkv, b_qkv, W_down, b_down, W_up, b_up):
    B, C, H, W = input.shape
    xu = input.reshape(B, C, _S, _P, _S, _P)
    xu = jnp.transpose(xu, (0, 1, 3, 5, 2, 4)).reshape(B, C * _P * _P, _S * _S)
    x = xu.reshape(B, _S * _S, -1, _P * _P)
    x = jnp.transpose(x, (0, 2, 1, 3)).reshape(B, -1, _S * _S)
    M = x.shape[1]
    nb = M // _BM

    xr = x.reshape(B, nb, _BM, _S * _S)
    z8 = jnp.zeros((B, 1, _HALO, _S * _S), x.dtype)
    fh = jnp.concatenate([z8, xr[:, :-1, _BM - _HALO:, :]], axis=1)
    bh = jnp.concatenate([xr[:, 1:, :_HALO, :], z8], axis=1)
    x01 = x[:, :_HALO, :]

    WT = W_qkv.T
    wq, wk, wv = WT[:, :_D3], WT[:, _D3:2 * _D3], WT[:, 2 * _D3:]
    bq = b_qkv[:_D3].reshape(1, _D3)
    bk = b_qkv[_D3:2 * _D3].reshape(1, _D3)
    bv = b_qkv[2 * _D3:].reshape(1, _D3)
    wd = jnp.transpose(W_down, (2, 1, 0))
    wu = jnp.transpose(W_up, (2, 1, 0))
    bd = b_down.reshape(1, _D3)
    bu = b_up.reshape(1, _D3)

    def full(shp, nd):
        return pl.BlockSpec(shp, (lambda b, i: (0,) * nd))

    out = pl.pallas_call(
        lambda *refs: _block_kernel(M, *refs),
        grid=(B, nb),
        in_specs=[
            pl.BlockSpec((1, _BM, _S * _S), lambda b, i: (b, i, 0)),
            pl.BlockSpec((1, 1, _HALO, _S * _S), lambda b, i: (b, i, 0, 0)),
            pl.BlockSpec((1, 1, _HALO, _S * _S), lambda b, i: (b, i, 0, 0)),
            pl.BlockSpec((1, _HALO, _S * _S), lambda b, i: (b, 0, 0)),
            full((_S * _S, _D3), 2),
            full((_S * _S, _D3), 2),
            full((_S * _S, _D3), 2),
            full((1, _D3), 2),
            full((1, _D3), 2),
            full((1, _D3), 2),
            full((3, _D3, _D3), 3),
            full((1, _D3), 2),
            full((3, _D3, _D3), 3),
            full((1, _D3), 2),
        ],
        out_specs=pl.BlockSpec((1, _BM, _D3), lambda b, i: (b, i, 0)),
        out_shape=jax.ShapeDtypeStruct((B, M, _D3), jnp.float32),
        compiler_params=pltpu.CompilerParams(
            dimension_semantics=("arbitrary", "arbitrary")),
    )(x, fh, bh, x01, wq, wk, wv, bq, bk, bv, wd, bd, wu, bu)

    out = out.reshape(B, -1, _S * _S, _P * _P)
    out = jnp.transpose(out, (0, 2, 1, 3))
    return out.reshape(B, -1, _S * _P, _S * _P)
